# baseline (speedup 1.0000x reference)
import functools

import jax
import jax.numpy as jnp
import numpy as np
from jax.experimental import pallas as pl
from jax.experimental.pallas import tpu as pltpu

_FOLD = np.array([[[1, 0, 0], [0, 1, 1]],
                  [[1, 1, 0], [0, 0, 1]]], dtype=np.float32)


def _fold_weights_t(w_oihw):
    fold = jnp.asarray(_FOLD)
    return jnp.einsum("apk,bql,oikl->abpqoi", fold, fold, w_oihw)


def _conv_body(x0_ref, x1_ref, x2_ref, w_ref, b_ref, o_ref, *, H, W, Cin, Cout):
    M = H * W
    bias_v = b_ref[...]

    xq = (x0_ref, x1_ref, x2_ref)
    win = {}
    for q in range(3):
        for p in range(3):
            win[(p, q)] = xq[q][0, :, p * W:p * W + M]

    for a in range(2):
        accs = []
        for b in range(2):
            acc = None
            for dy in range(2):
                for dx in range(2):
                    d = jnp.dot(w_ref[a, b, dy, dx], win[(a + dy, b + dx)],
                                preferred_element_type=jnp.float32)
                    acc = d if acc is None else acc + d
            accs.append(acc + bias_v)
        o_ref[0, a] = pltpu.pack_elementwise(accs, packed_dtype=jnp.bfloat16)


def kernel(x_nchw, conv_weight_oihw, conv_bias):
    N, C, H, W = x_nchw.shape
    Cout = conv_weight_oihw.shape[0]
    M = H * W

    xb = x_nchw.astype(jnp.bfloat16)
    zero = ((0, 0), (0, 0))
    slab_list = [
        jnp.pad(xb[:, :, :, :W - 1], (*zero, (1, 1), (1, 0))),
        jnp.pad(xb, (*zero, (1, 1), (0, 0))),
        jnp.pad(xb[:, :, :, 1:], (*zero, (1, 1), (0, 1))),
    ]
    slabs = [s.reshape(N, C, (H + 2) * W) for s in slab_list]

    w_t = _fold_weights_t(conv_weight_oihw).astype(jnp.bfloat16)
    bias2 = conv_bias.reshape(Cout, 1).astype(jnp.float32)

    body = functools.partial(_conv_body, H=H, W=W, Cin=C, Cout=Cout)
    xq_spec = pl.BlockSpec((1, C, (H + 2) * W), lambda n: (n, 0, 0))
    y_packed = pl.pallas_call(
        body,
        out_shape=jax.ShapeDtypeStruct((N, 2, Cout, M), jnp.int32),
        grid=(N,),
        in_specs=[
            xq_spec, xq_spec, xq_spec,
            pl.BlockSpec((2, 2, 2, 2, Cout, C), lambda n: (0, 0, 0, 0, 0, 0)),
            pl.BlockSpec((Cout, 1), lambda n: (0, 0)),
        ],
        out_specs=pl.BlockSpec((1, 2, Cout, M), lambda n: (n, 0, 0, 0)),
        compiler_params=pltpu.CompilerParams(
            dimension_semantics=("parallel",)),
        cost_estimate=pl.CostEstimate(
            flops=int(2 * 16 * N * M * C * Cout),
            transcendentals=0,
            bytes_accessed=int(N * C * (3 * (H + 2) * W * 2 + 2 * M * 4)),
        ),
    )(*slabs, w_t, bias2)

    yz = jnp.transpose(y_packed.reshape(N, 2, Cout, H, W), (0, 2, 3, 1, 4))
    yb = jax.lax.bitcast_convert_type(yz, jnp.bfloat16)
    y = yb.astype(jnp.float32)
    return y.reshape(N, Cout, 2 * H, 2 * W)


# speedup vs baseline: 1.1101x; 1.1101x over previous
import functools

import jax
import jax.numpy as jnp
import numpy as np
from jax.experimental import pallas as pl
from jax.experimental.pallas import tpu as pltpu

_FOLD = np.array([[[1, 0, 0], [0, 1, 1]],
                  [[1, 1, 0], [0, 0, 1]]], dtype=np.float32)


def _fold_weights_t(w_oihw):
    fold = jnp.asarray(_FOLD)
    return jnp.einsum("apk,bql,oikl->abpqoi", fold, fold, w_oihw)


def _conv_body(x0_ref, x1_ref, x2_ref, w_ref, b_ref, o_ref, *, H, W, Cin, Cout):
    M = H * W
    bias_v = b_ref[...]

    xq = (x0_ref, x1_ref, x2_ref)
    win = {}
    for q in range(3):
        for p in range(3):
            win[(p, q)] = xq[q][0, :, p * W:p * W + M]

    packed = []
    for a in range(2):
        accs = []
        for b in range(2):
            acc = None
            for dy in range(2):
                for dx in range(2):
                    d = jnp.dot(w_ref[a, b, dy, dx], win[(a + dy, b + dx)],
                                preferred_element_type=jnp.float32)
                    acc = d if acc is None else acc + d
            accs.append(acc + bias_v)
        packed.append(pltpu.pack_elementwise(accs, packed_dtype=jnp.bfloat16))

    parts = []
    for i in range(H):
        parts.append(packed[0][:, i * W:(i + 1) * W])
        parts.append(packed[1][:, i * W:(i + 1) * W])
    o_ref[0] = jnp.concatenate(parts, axis=-1)


def kernel(x_nchw, conv_weight_oihw, conv_bias):
    N, C, H, W = x_nchw.shape
    Cout = conv_weight_oihw.shape[0]
    M = H * W

    xpad = jnp.pad(x_nchw.astype(jnp.bfloat16),
                   ((0, 0), (0, 0), (1, 1), (1, 1)))
    slabs = [xpad[:, :, :, q:q + W].reshape(N, C, (H + 2) * W)
             for q in range(3)]

    w_t = _fold_weights_t(conv_weight_oihw).astype(jnp.bfloat16)
    bias2 = conv_bias.reshape(Cout, 1).astype(jnp.float32)

    body = functools.partial(_conv_body, H=H, W=W, Cin=C, Cout=Cout)
    xq_spec = pl.BlockSpec((1, C, (H + 2) * W), lambda n: (n, 0, 0))
    y_packed = pl.pallas_call(
        body,
        out_shape=jax.ShapeDtypeStruct((N, Cout, 2 * M), jnp.int32),
        grid=(N,),
        in_specs=[
            xq_spec, xq_spec, xq_spec,
            pl.BlockSpec((2, 2, 2, 2, Cout, C), lambda n: (0, 0, 0, 0, 0, 0)),
            pl.BlockSpec((Cout, 1), lambda n: (0, 0)),
        ],
        out_specs=pl.BlockSpec((1, Cout, 2 * M), lambda n: (n, 0, 0)),
        compiler_params=pltpu.CompilerParams(
            dimension_semantics=("parallel",)),
        cost_estimate=pl.CostEstimate(
            flops=int(2 * 16 * N * M * C * Cout),
            transcendentals=0,
            bytes_accessed=int(N * C * (3 * (H + 2) * W * 2 + 2 * M * 4)),
        ),
    )(*slabs, w_t, bias2)

    yb = jax.lax.bitcast_convert_type(y_packed, jnp.bfloat16)
    y = yb.astype(jnp.float32)
    return y.reshape(N, Cout, 2 * H, 2 * W)


# speedup vs baseline: 1.3310x; 1.1989x over previous
import functools

import jax
import jax.numpy as jnp
import numpy as np
from jax.experimental import pallas as pl
from jax.experimental.pallas import tpu as pltpu

_FOLD = np.array([[[1, 0, 0], [0, 1, 1]],
                  [[1, 1, 0], [0, 0, 1]]], dtype=np.float32)


def _fold_weights_t(w_oihw):
    fold = jnp.asarray(_FOLD)
    return jnp.einsum("apk,bql,oikl->abpqoi", fold, fold, w_oihw)


def _conv_body(x_ref, w_ref, b_ref, o_ref, *, H, W, Cin, Cout):
    M = H * W
    bias_v = b_ref[...]
    xf = x_ref[0]

    lane = jax.lax.broadcasted_iota(jnp.int32, (Cin, M), 1)
    col = jax.lax.rem(lane, W)

    bufs = []
    for q in range(3):
        if q == 1:
            v = xf
        else:
            v = pltpu.roll(xf, (1 - q) % M, axis=1)
            edge = 0 if q == 0 else W - 1
            v = jnp.where(col == edge, 0.0, v)
        vb = v.astype(jnp.bfloat16)
        zpad = jnp.zeros((Cin, W), jnp.bfloat16)
        bufs.append(jnp.concatenate([zpad, vb, zpad], axis=1))

    win = {}
    for q in range(3):
        for p in range(3):
            win[(p, q)] = bufs[q][:, p * W:p * W + M]

    packed = []
    for a in range(2):
        accs = []
        for b in range(2):
            acc = None
            for dy in range(2):
                for dx in range(2):
                    d = jnp.dot(w_ref[a, b, dy, dx], win[(a + dy, b + dx)],
                                preferred_element_type=jnp.float32)
                    acc = d if acc is None else acc + d
            accs.append(acc + bias_v)
        packed.append(pltpu.pack_elementwise(accs, packed_dtype=jnp.bfloat16))

    parts = []
    for i in range(H):
        parts.append(packed[0][:, i * W:(i + 1) * W])
        parts.append(packed[1][:, i * W:(i + 1) * W])
    o_ref[0] = jnp.concatenate(parts, axis=-1)


def kernel(x_nchw, conv_weight_oihw, conv_bias):
    N, C, H, W = x_nchw.shape
    Cout = conv_weight_oihw.shape[0]
    M = H * W

    x_flat = x_nchw.reshape(N, C, M)
    w_t = _fold_weights_t(conv_weight_oihw).astype(jnp.bfloat16)
    bias2 = conv_bias.reshape(Cout, 1).astype(jnp.float32)

    body = functools.partial(_conv_body, H=H, W=W, Cin=C, Cout=Cout)
    y_packed = pl.pallas_call(
        body,
        out_shape=jax.ShapeDtypeStruct((N, Cout, 2 * M), jnp.int32),
        grid=(N,),
        in_specs=[
            pl.BlockSpec((1, C, M), lambda n: (n, 0, 0)),
            pl.BlockSpec((2, 2, 2, 2, Cout, C), lambda n: (0, 0, 0, 0, 0, 0)),
            pl.BlockSpec((Cout, 1), lambda n: (0, 0)),
        ],
        out_specs=pl.BlockSpec((1, Cout, 2 * M), lambda n: (n, 0, 0)),
        compiler_params=pltpu.CompilerParams(
            dimension_semantics=("parallel",)),
        cost_estimate=pl.CostEstimate(
            flops=int(2 * 16 * N * M * C * Cout),
            transcendentals=0,
            bytes_accessed=int(N * C * (M * 4 + 2 * M * 4)),
        ),
    )(x_flat, w_t, bias2)

    even_bits = y_packed << 16
    odd_bits = y_packed & jnp.int32(-65536)
    pair = jnp.stack([even_bits, odd_bits], axis=-1)
    y = jax.lax.bitcast_convert_type(pair, jnp.float32)
    return y.reshape(N, Cout, 2 * H, 2 * W)


# speedup vs baseline: 2.1112x; 1.5862x over previous
import functools

import jax
import jax.numpy as jnp
import numpy as np
from jax.experimental import pallas as pl
from jax.experimental.pallas import tpu as pltpu

_FOLD = np.array([[[1, 0, 0], [0, 1, 1]],
                  [[1, 1, 0], [0, 0, 1]]], dtype=np.float32)


def _fold_weights(w_oihw):
    fold = jnp.asarray(_FOLD)
    return jnp.einsum("apk,bql,oikl->abpqio", fold, fold, w_oihw)


def _conv_body(xp_ref, w_ref, b_ref, o_ref, *, TH, W, Cin, Cout):
    M = TH * W
    bias_v = b_ref[...]

    win = {}
    for p in range(3):
        for q in range(3):
            win[(p, q)] = xp_ref[0, pl.ds(p, TH), pl.ds(q, W), :].reshape(M, Cin)

    for a in range(2):
        for b in range(2):
            acc = None
            for dy in range(2):
                for dx in range(2):
                    d = jnp.dot(win[(a + dy, b + dx)], w_ref[a, b, dy, dx],
                                preferred_element_type=jnp.float32)
                    acc = d if acc is None else acc + d
            acc = acc + bias_v
            o_ref[0, a, b] = acc.reshape(TH, W, Cout).astype(o_ref.dtype)


def kernel(x_nchw, conv_weight_oihw, conv_bias):
    N, C, H, W = x_nchw.shape
    Cout = conv_weight_oihw.shape[0]
    TH = H

    x = jnp.transpose(x_nchw, (0, 2, 3, 1))
    x_pad = jnp.pad(x, ((0, 0), (1, 1), (1, 1), (0, 0))).astype(jnp.bfloat16)
    w_eff = _fold_weights(conv_weight_oihw).astype(jnp.bfloat16)
    bias2 = conv_bias.reshape(1, Cout).astype(jnp.float32)

    body = functools.partial(_conv_body, TH=TH, W=W, Cin=C, Cout=Cout)
    y_sub = pl.pallas_call(
        body,
        out_shape=jax.ShapeDtypeStruct((N, 2, 2, H, W, Cout), jnp.float32),
        grid=(N,),
        in_specs=[
            pl.BlockSpec((1, H + 2, W + 2, C), lambda n: (n, 0, 0, 0)),
            pl.BlockSpec((2, 2, 2, 2, C, Cout), lambda n: (0, 0, 0, 0, 0, 0)),
            pl.BlockSpec((1, Cout), lambda n: (0, 0)),
        ],
        out_specs=pl.BlockSpec((1, 2, 2, TH, W, Cout), lambda n: (n, 0, 0, 0, 0, 0)),
        compiler_params=pltpu.CompilerParams(
            dimension_semantics=("parallel",)),
        cost_estimate=pl.CostEstimate(
            flops=int(2 * 16 * N * H * W * C * Cout),
            transcendentals=0,
            bytes_accessed=int(N * (H + 2) * (W + 2) * C * 2
                               + N * 4 * H * W * Cout * 4),
        ),
    )(x_pad, w_eff, bias2)

    y = jnp.transpose(y_sub, (0, 5, 3, 1, 4, 2))
    return y.reshape(N, Cout, 2 * H, 2 * W)
